# baseline (speedup 1.0000x reference)
import jax, jax.numpy as jnp
from jax.experimental import pallas as pl
from jax.experimental.pallas import tpu as pltpu

DROP_FRAME_PROB = 0.125


def _gather_body(src_ref, in_ref, out_ref):
    out_ref[...] = in_ref[...]


def kernel(frames, mask):
    T = frames.shape[0]
    row = frames.size // T
    f2 = frames.reshape(T, row // 128, 128)

    rkey = jax.random.key(42)
    kdrop, kdir = jax.random.split(rkey)
    u_drop = jax.random.uniform(kdrop, (T,))
    u_dir = jax.random.uniform(kdir, (T,))
    drop = u_drop < DROP_FRAME_PROB
    diff = jnp.where(u_dir < 0.5, -1, 1)
    idx = jnp.arange(T)
    src = jnp.where(drop, (idx + diff) % T, idx).astype(jnp.int32)

    grid_spec = pltpu.PrefetchScalarGridSpec(
        num_scalar_prefetch=1,
        grid=(T,),
        in_specs=[pl.BlockSpec((1, row // 128, 128),
                               lambda i, src_ref: (src_ref[i], 0, 0))],
        out_specs=pl.BlockSpec((1, row // 128, 128),
                               lambda i, src_ref: (i, 0, 0)),
    )
    out = pl.pallas_call(
        _gather_body,
        grid_spec=grid_spec,
        out_shape=jax.ShapeDtypeStruct(f2.shape, f2.dtype),
    )(src, f2)
    return (out.reshape(frames.shape), mask)
